# Initial kernel scaffold; baseline (speedup 1.0000x reference)
import jax
import jax.numpy as jnp
from jax.experimental import pallas as pl


def kernel(x_lesion, edge_index, batch_lesion, W_self1, W_neigh1, W_self2, W_neigh2, W1, b1, W2, b2):
    raise NotImplementedError("write your pallas kernel here")


# speedup vs baseline: 6.8059x; 6.8059x over previous
import functools

import jax
import jax.numpy as jnp
from jax import lax
from jax.experimental import pallas as pl
from jax.experimental.pallas import tpu as pltpu
from jax.experimental.pallas import tpu_sc as plsc

N = 10000
NP = 10240
E = 320000
D = 128
G = 64
H = 64
C = 2

NC = 2
NS = 16
K = 80
TILE_EDGES = E // (NC * NS)
CHUNKS = TILE_EDGES // K
RPS = NP // NS
BLK = 1024
NBLK = NP // BLK

_PREC = lax.Precision.HIGHEST


def _sc_agg_body(x_hbm, src_hbm, dst_hbm, zeros_hbm,
                 p_ref, src_v, dst_v, rows_v, sem, acc):
    c = lax.axis_index("c")
    s = lax.axis_index("s")
    wid = c * NS + s
    r0 = s * RPS
    pltpu.sync_copy(zeros_hbm.at[pl.ds(r0, RPS)], acc.at[pl.ds(r0, RPS)])
    pltpu.sync_copy(src_hbm.at[wid], src_v)
    pltpu.sync_copy(dst_hbm.at[wid], dst_v)
    plsc.subcore_barrier()

    def chunk(j, carry):
        pltpu.async_copy(x_hbm.at[src_v.at[j]], rows_v, sem).wait()
        pltpu.sync_copy(rows_v, acc.at[dst_v.at[j]], add=True)
        return carry

    lax.fori_loop(0, CHUNKS, chunk, 0)
    plsc.subcore_barrier()
    pltpu.sync_copy(acc.at[pl.ds(r0, RPS)], p_ref.at[c, pl.ds(r0, RPS)])


def _sc_cnt_body(dst_hbm, zeros_hbm, ones_hbm,
                 cnt_ref, dst_v, ones_v, cntacc):
    c = lax.axis_index("c")
    s = lax.axis_index("s")
    wid = c * NS + s
    r0 = s * RPS
    pltpu.sync_copy(zeros_hbm.at[pl.ds(r0, RPS)], cntacc.at[pl.ds(r0, RPS)])
    pltpu.sync_copy(ones_hbm, ones_v)
    pltpu.sync_copy(dst_hbm.at[wid], dst_v)
    plsc.subcore_barrier()

    def chunk(j, carry):
        pltpu.sync_copy(ones_v, cntacc.at[dst_v.at[j]], add=True)
        return carry

    lax.fori_loop(0, CHUNKS, chunk, 0)
    plsc.subcore_barrier()
    pltpu.sync_copy(cntacc.at[pl.ds(r0, RPS)], cnt_ref.at[c, pl.ds(r0, RPS)])


def _make_sc_agg():
    mesh = plsc.VectorSubcoreMesh(core_axis_name="c", subcore_axis_name="s")
    return pl.kernel(
        _sc_agg_body,
        out_type=jax.ShapeDtypeStruct((NC, NP, D), jnp.float32),
        mesh=mesh,
        scratch_types=[
            pltpu.VMEM((CHUNKS, K), jnp.int32),
            pltpu.VMEM((CHUNKS, K), jnp.int32),
            pltpu.VMEM((K, D), jnp.float32),
            pltpu.SemaphoreType.DMA,
            pltpu.VMEM_SHARED((NP, D), jnp.float32),
        ],
    )


def _make_sc_cnt():
    mesh = plsc.VectorSubcoreMesh(core_axis_name="c", subcore_axis_name="s")
    return pl.kernel(
        _sc_cnt_body,
        out_type=jax.ShapeDtypeStruct((NC, NP, D), jnp.float32),
        mesh=mesh,
        scratch_types=[
            pltpu.VMEM((CHUNKS, K), jnp.int32),
            pltpu.VMEM((K, D), jnp.float32),
            pltpu.VMEM_SHARED((NP, D), jnp.float32),
        ],
    )


def _combine_body(x_ref, p_ref, cnt_ref, ws_ref, wn_ref, o_ref):
    ssum = p_ref[0] + p_ref[1]
    cntv = cnt_ref[0, :, 0:1] + cnt_ref[1, :, 0:1]
    agg = ssum / jnp.maximum(cntv, 1.0)
    h = (jnp.dot(x_ref[...], ws_ref[...], precision=_PREC,
                 preferred_element_type=jnp.float32)
         + jnp.dot(agg, wn_ref[...], precision=_PREC,
                   preferred_element_type=jnp.float32))
    o_ref[...] = jnp.where(h >= 0.0, h, 0.01 * h)


def _combine1(x, p, cnt, ws, wn):
    return pl.pallas_call(
        _combine_body,
        grid=(NBLK,),
        in_specs=[
            pl.BlockSpec((BLK, D), lambda i: (i, 0)),
            pl.BlockSpec((NC, BLK, D), lambda i: (0, i, 0)),
            pl.BlockSpec((NC, BLK, D), lambda i: (0, i, 0)),
            pl.BlockSpec((D, D), lambda i: (0, 0)),
            pl.BlockSpec((D, D), lambda i: (0, 0)),
        ],
        out_specs=pl.BlockSpec((BLK, D), lambda i: (i, 0)),
        out_shape=jax.ShapeDtypeStruct((NP, D), jnp.float32),
    )(x, p, cnt, ws, wn)


def _final_body(x_ref, p_ref, cnt_ref, ws_ref, wn_ref, batch_ref,
                w1_ref, b1_ref, w2_ref, b2_ref, o_ref, pacc, cacc):
    i = pl.program_id(0)
    ssum = p_ref[0] + p_ref[1]
    cntv = cnt_ref[0, :, 0:1] + cnt_ref[1, :, 0:1]
    agg = ssum / jnp.maximum(cntv, 1.0)
    h = (jnp.dot(x_ref[...], ws_ref[...], precision=_PREC,
                 preferred_element_type=jnp.float32)
         + jnp.dot(agg, wn_ref[...], precision=_PREC,
                   preferred_element_type=jnp.float32))
    h = jnp.where(h >= 0.0, h, 0.01 * h)
    ids = batch_ref[0]
    io = lax.broadcasted_iota(jnp.int32, (G, BLK), 0)
    oh = (io == ids).astype(jnp.float32)

    @pl.when(i == 0)
    def _():
        pacc[...] = jnp.zeros_like(pacc)
        cacc[...] = jnp.zeros_like(cacc)

    pacc[...] += jnp.dot(oh, h, precision=_PREC,
                         preferred_element_type=jnp.float32)
    cacc[...] += jnp.dot(oh, jnp.ones((BLK, D), jnp.float32), precision=_PREC,
                         preferred_element_type=jnp.float32)

    @pl.when(i == NBLK - 1)
    def _():
        pooled = pacc[...] / jnp.maximum(cacc[...], 1.0)
        t = jnp.dot(pooled, w1_ref[...], precision=_PREC,
                    preferred_element_type=jnp.float32) + b1_ref[...]
        t = jnp.maximum(t, 0.0)
        o_ref[...] = jnp.dot(t, w2_ref[...], precision=_PREC,
                             preferred_element_type=jnp.float32) + b2_ref[...]


def _final(x, p, cnt, ws, wn, batch3, w1, b1, w2, b2):
    return pl.pallas_call(
        _final_body,
        grid=(NBLK,),
        in_specs=[
            pl.BlockSpec((BLK, D), lambda i: (i, 0)),
            pl.BlockSpec((NC, BLK, D), lambda i: (0, i, 0)),
            pl.BlockSpec((NC, BLK, D), lambda i: (0, i, 0)),
            pl.BlockSpec((D, D), lambda i: (0, 0)),
            pl.BlockSpec((D, D), lambda i: (0, 0)),
            pl.BlockSpec((1, 1, BLK), lambda i: (i, 0, 0)),
            pl.BlockSpec((D, H), lambda i: (0, 0)),
            pl.BlockSpec((1, H), lambda i: (0, 0)),
            pl.BlockSpec((H, C), lambda i: (0, 0)),
            pl.BlockSpec((1, C), lambda i: (0, 0)),
        ],
        out_specs=pl.BlockSpec((G, C), lambda i: (0, 0)),
        out_shape=jax.ShapeDtypeStruct((G, C), jnp.float32),
        scratch_shapes=[
            pltpu.VMEM((G, D), jnp.float32),
            pltpu.VMEM((G, D), jnp.float32),
        ],
    )(x, p, cnt, ws, wn, batch3, w1, b1, w2, b2)


def kernel(x_lesion, edge_index, batch_lesion,
           W_self1, W_neigh1, W_self2, W_neigh2, W1, b1, W2, b2):
    x = jnp.concatenate(
        [x_lesion, jnp.zeros((NP - N, D), jnp.float32)], axis=0)
    src = edge_index[0].astype(jnp.int32).reshape(NC * NS, CHUNKS, K)
    dst = edge_index[1].astype(jnp.int32).reshape(NC * NS, CHUNKS, K)
    batch3 = jnp.concatenate(
        [batch_lesion.astype(jnp.int32),
         jnp.full((NP - N,), G, jnp.int32)]).reshape(NBLK, 1, BLK)
    zeros = jnp.zeros((NP, D), jnp.float32)
    ones = jnp.ones((K, D), jnp.float32)

    cnt = _make_sc_cnt()(dst, zeros, ones)
    p1 = _make_sc_agg()(x, src, dst, zeros)
    h2 = _combine1(x, p1, cnt, W_self1, W_neigh1)
    p2 = _make_sc_agg()(h2, src, dst, zeros)
    return _final(h2, p2, cnt, W_self2, W_neigh2, batch3,
                  W1, b1.reshape(1, H), W2, b2.reshape(1, C))


# speedup vs baseline: 7.6053x; 1.1175x over previous
import functools

import jax
import jax.numpy as jnp
from jax import lax
from jax.experimental import pallas as pl
from jax.experimental.pallas import tpu as pltpu
from jax.experimental.pallas import tpu_sc as plsc

N = 10000
NP = 10240
E = 320000
D = 128
G = 64
H = 64
C = 2

NC = 2
NS = 16
K = 125
TILE_EDGES = E // (NC * NS)
CHUNKS = TILE_EDGES // K
RPS = NP // NS
BLK = 1024
NBLK = NP // BLK

_PREC = lax.Precision.HIGHEST


def _sc_agg_body(x_hbm, src_hbm, dst_hbm, zeros_hbm,
                 p_ref, src_v, dst_v, rows2, gsem, acc):
    c = lax.axis_index("c")
    s = lax.axis_index("s")
    wid = c * NS + s
    r0 = s * RPS
    pltpu.sync_copy(zeros_hbm.at[pl.ds(r0, RPS)], acc.at[pl.ds(r0, RPS)])
    pltpu.sync_copy(src_hbm.at[wid], src_v)
    pltpu.sync_copy(dst_hbm.at[wid], dst_v)
    plsc.subcore_barrier()

    def step(j, carry):
        pltpu.async_copy(x_hbm.at[src_v.at[j]], rows2, gsem).wait()
        pltpu.sync_copy(rows2, acc.at[dst_v.at[j]], add=True)
        return carry

    lax.fori_loop(0, CHUNKS, step, 0)
    plsc.subcore_barrier()
    pltpu.sync_copy(acc.at[pl.ds(r0, RPS)], p_ref.at[c, pl.ds(r0, RPS)])


def _sc_cnt_body(dst_hbm, zeros_hbm, ones_hbm,
                 cnt_ref, dst_v, ones_v, cntacc):
    c = lax.axis_index("c")
    s = lax.axis_index("s")
    wid = c * NS + s
    r0 = s * RPS
    pltpu.sync_copy(zeros_hbm.at[pl.ds(r0, RPS)], cntacc.at[pl.ds(r0, RPS)])
    pltpu.sync_copy(ones_hbm, ones_v)
    pltpu.sync_copy(dst_hbm.at[wid], dst_v)
    plsc.subcore_barrier()

    def chunk(j, carry):
        pltpu.sync_copy(ones_v, cntacc.at[dst_v.at[j]], add=True)
        return carry

    lax.fori_loop(0, CHUNKS, chunk, 0)
    plsc.subcore_barrier()
    pltpu.sync_copy(cntacc.at[pl.ds(r0, RPS)], cnt_ref.at[c, pl.ds(r0, RPS)])


def _make_sc_agg():
    mesh = plsc.VectorSubcoreMesh(core_axis_name="c", subcore_axis_name="s")
    return pl.kernel(
        _sc_agg_body,
        out_type=jax.ShapeDtypeStruct((NC, NP, D), jnp.float32),
        mesh=mesh,
        scratch_types=[
            pltpu.VMEM((CHUNKS, K), jnp.int32),
            pltpu.VMEM((CHUNKS, K), jnp.int32),
            pltpu.VMEM((K, D), jnp.float32),
            pltpu.SemaphoreType.DMA,
            pltpu.VMEM_SHARED((NP, D), jnp.float32),
        ],
    )


def _make_sc_cnt():
    mesh = plsc.VectorSubcoreMesh(core_axis_name="c", subcore_axis_name="s")
    return pl.kernel(
        _sc_cnt_body,
        out_type=jax.ShapeDtypeStruct((NC, NP, D), jnp.float32),
        mesh=mesh,
        scratch_types=[
            pltpu.VMEM((CHUNKS, K), jnp.int32),
            pltpu.VMEM((K, D), jnp.float32),
            pltpu.VMEM_SHARED((NP, D), jnp.float32),
        ],
    )


def _combine_body(x_ref, p_ref, cnt_ref, ws_ref, wn_ref, o_ref):
    ssum = p_ref[0] + p_ref[1]
    cntv = cnt_ref[0, :, 0:1] + cnt_ref[1, :, 0:1]
    agg = ssum / jnp.maximum(cntv, 1.0)
    h = (jnp.dot(x_ref[...], ws_ref[...], precision=_PREC,
                 preferred_element_type=jnp.float32)
         + jnp.dot(agg, wn_ref[...], precision=_PREC,
                   preferred_element_type=jnp.float32))
    o_ref[...] = jnp.where(h >= 0.0, h, 0.01 * h)


def _combine1(x, p, cnt, ws, wn):
    return pl.pallas_call(
        _combine_body,
        grid=(NBLK,),
        in_specs=[
            pl.BlockSpec((BLK, D), lambda i: (i, 0)),
            pl.BlockSpec((NC, BLK, D), lambda i: (0, i, 0)),
            pl.BlockSpec((NC, BLK, D), lambda i: (0, i, 0)),
            pl.BlockSpec((D, D), lambda i: (0, 0)),
            pl.BlockSpec((D, D), lambda i: (0, 0)),
        ],
        out_specs=pl.BlockSpec((BLK, D), lambda i: (i, 0)),
        out_shape=jax.ShapeDtypeStruct((NP, D), jnp.float32),
    )(x, p, cnt, ws, wn)


def _final_body(x_ref, p_ref, cnt_ref, ws_ref, wn_ref, batch_ref,
                w1_ref, b1_ref, w2_ref, b2_ref, o_ref, pacc, cacc):
    i = pl.program_id(0)
    ssum = p_ref[0] + p_ref[1]
    cntv = cnt_ref[0, :, 0:1] + cnt_ref[1, :, 0:1]
    agg = ssum / jnp.maximum(cntv, 1.0)
    h = (jnp.dot(x_ref[...], ws_ref[...], precision=_PREC,
                 preferred_element_type=jnp.float32)
         + jnp.dot(agg, wn_ref[...], precision=_PREC,
                   preferred_element_type=jnp.float32))
    h = jnp.where(h >= 0.0, h, 0.01 * h)
    ids = batch_ref[0]
    io = lax.broadcasted_iota(jnp.int32, (G, BLK), 0)
    oh = (io == ids).astype(jnp.float32)

    @pl.when(i == 0)
    def _():
        pacc[...] = jnp.zeros_like(pacc)
        cacc[...] = jnp.zeros_like(cacc)

    pacc[...] += jnp.dot(oh, h, precision=_PREC,
                         preferred_element_type=jnp.float32)
    cacc[...] += jnp.dot(oh, jnp.ones((BLK, D), jnp.float32), precision=_PREC,
                         preferred_element_type=jnp.float32)

    @pl.when(i == NBLK - 1)
    def _():
        pooled = pacc[...] / jnp.maximum(cacc[...], 1.0)
        t = jnp.dot(pooled, w1_ref[...], precision=_PREC,
                    preferred_element_type=jnp.float32) + b1_ref[...]
        t = jnp.maximum(t, 0.0)
        o_ref[...] = jnp.dot(t, w2_ref[...], precision=_PREC,
                             preferred_element_type=jnp.float32) + b2_ref[...]


def _final(x, p, cnt, ws, wn, batch3, w1, b1, w2, b2):
    return pl.pallas_call(
        _final_body,
        grid=(NBLK,),
        in_specs=[
            pl.BlockSpec((BLK, D), lambda i: (i, 0)),
            pl.BlockSpec((NC, BLK, D), lambda i: (0, i, 0)),
            pl.BlockSpec((NC, BLK, D), lambda i: (0, i, 0)),
            pl.BlockSpec((D, D), lambda i: (0, 0)),
            pl.BlockSpec((D, D), lambda i: (0, 0)),
            pl.BlockSpec((1, 1, BLK), lambda i: (i, 0, 0)),
            pl.BlockSpec((D, H), lambda i: (0, 0)),
            pl.BlockSpec((1, H), lambda i: (0, 0)),
            pl.BlockSpec((H, C), lambda i: (0, 0)),
            pl.BlockSpec((1, C), lambda i: (0, 0)),
        ],
        out_specs=pl.BlockSpec((G, C), lambda i: (0, 0)),
        out_shape=jax.ShapeDtypeStruct((G, C), jnp.float32),
        scratch_shapes=[
            pltpu.VMEM((G, D), jnp.float32),
            pltpu.VMEM((G, D), jnp.float32),
        ],
    )(x, p, cnt, ws, wn, batch3, w1, b1, w2, b2)


def kernel(x_lesion, edge_index, batch_lesion,
           W_self1, W_neigh1, W_self2, W_neigh2, W1, b1, W2, b2):
    x = jnp.concatenate(
        [x_lesion, jnp.zeros((NP - N, D), jnp.float32)], axis=0)
    src = edge_index[0].astype(jnp.int32).reshape(NC * NS, CHUNKS, K)
    dst = edge_index[1].astype(jnp.int32).reshape(NC * NS, CHUNKS, K)
    batch3 = jnp.concatenate(
        [batch_lesion.astype(jnp.int32),
         jnp.full((NP - N,), G, jnp.int32)]).reshape(NBLK, 1, BLK)
    zeros = jnp.zeros((NP, D), jnp.float32)
    ones = jnp.ones((K, D), jnp.float32)

    cnt = _make_sc_cnt()(dst, zeros, ones)
    p1 = _make_sc_agg()(x, src, dst, zeros)
    h2 = _combine1(x, p1, cnt, W_self1, W_neigh1)
    p2 = _make_sc_agg()(h2, src, dst, zeros)
    return _final(h2, p2, cnt, W_self2, W_neigh2, batch3,
                  W1, b1.reshape(1, H), W2, b2.reshape(1, C))


# speedup vs baseline: 7.6258x; 1.0027x over previous
import functools

import jax
import jax.numpy as jnp
from jax import lax
from jax.experimental import pallas as pl
from jax.experimental.pallas import tpu as pltpu
from jax.experimental.pallas import tpu_sc as plsc

N = 10000
NP = 10240
E = 320000
D = 128
G = 64
H = 64
C = 2

NC = 2
NS = 16
K = 125
TILE_EDGES = E // (NC * NS)
CHUNKS = TILE_EDGES // K
RPS = NP // NS
BLK = 1024
NBLK = NP // BLK

_PREC = lax.Precision.HIGHEST


def _sc_agg_body(x_hbm, src_hbm, dst_hbm, zeros_hbm,
                 p_ref, src_v, dst_v, rows2, gsem, acc):
    c = lax.axis_index("c")
    s = lax.axis_index("s")
    wid = c * NS + s
    r0 = s * RPS
    pltpu.sync_copy(zeros_hbm.at[pl.ds(r0, RPS)], acc.at[pl.ds(r0, RPS)])
    pltpu.sync_copy(src_hbm.at[wid], src_v)
    pltpu.sync_copy(dst_hbm.at[wid], dst_v)
    plsc.subcore_barrier()

    def step(j, carry):
        pltpu.async_copy(x_hbm.at[src_v.at[j]], rows2, gsem).wait()
        pltpu.sync_copy(rows2, acc.at[dst_v.at[j]], add=True)
        return carry

    lax.fori_loop(0, CHUNKS, step, 0)
    plsc.subcore_barrier()
    pltpu.sync_copy(acc.at[pl.ds(r0, RPS)], p_ref.at[c, pl.ds(r0, RPS)])


def _sc_cnt_body(dst_hbm, zeros_hbm, ones_hbm,
                 cnt_ref, dst_v, ones_v, csem, cntacc):
    c = lax.axis_index("c")
    s = lax.axis_index("s")
    wid = c * NS + s
    r0 = s * RPS
    pltpu.sync_copy(zeros_hbm.at[pl.ds(r0, RPS)], cntacc.at[pl.ds(r0, RPS)])
    pltpu.sync_copy(ones_hbm, ones_v)
    pltpu.sync_copy(dst_hbm.at[wid], dst_v)
    plsc.subcore_barrier()

    def chunk(j, carry):
        @pl.when(j < CHUNKS)
        def _():
            pltpu.async_copy(ones_v, cntacc.at[dst_v.at[j]], csem, add=True)

        @pl.when(j >= 2)
        def _():
            pltpu.make_async_copy(ones_v, cntacc.at[dst_v.at[0]], csem).wait()

        return carry

    lax.fori_loop(0, CHUNKS + 2, chunk, 0)
    plsc.subcore_barrier()
    pltpu.sync_copy(cntacc.at[pl.ds(r0, RPS)], cnt_ref.at[c, pl.ds(r0, RPS)])


def _make_sc_agg():
    mesh = plsc.VectorSubcoreMesh(core_axis_name="c", subcore_axis_name="s")
    return pl.kernel(
        _sc_agg_body,
        out_type=jax.ShapeDtypeStruct((NC, NP, D), jnp.float32),
        mesh=mesh,
        scratch_types=[
            pltpu.VMEM((CHUNKS, K), jnp.int32),
            pltpu.VMEM((CHUNKS, K), jnp.int32),
            pltpu.VMEM((K, D), jnp.float32),
            pltpu.SemaphoreType.DMA,
            pltpu.VMEM_SHARED((NP, D), jnp.float32),
        ],
    )


def _make_sc_cnt():
    mesh = plsc.VectorSubcoreMesh(core_axis_name="c", subcore_axis_name="s")
    return pl.kernel(
        _sc_cnt_body,
        out_type=jax.ShapeDtypeStruct((NC, NP, D), jnp.float32),
        mesh=mesh,
        scratch_types=[
            pltpu.VMEM((CHUNKS, K), jnp.int32),
            pltpu.VMEM((K, D), jnp.float32),
            pltpu.SemaphoreType.DMA,
            pltpu.VMEM_SHARED((NP, D), jnp.float32),
        ],
    )


def _combine_body(x_ref, p_ref, cnt_ref, ws_ref, wn_ref, o_ref):
    ssum = p_ref[0] + p_ref[1]
    cntv = cnt_ref[0, :, 0:1] + cnt_ref[1, :, 0:1]
    agg = ssum / jnp.maximum(cntv, 1.0)
    h = (jnp.dot(x_ref[...], ws_ref[...], precision=_PREC,
                 preferred_element_type=jnp.float32)
         + jnp.dot(agg, wn_ref[...], precision=_PREC,
                   preferred_element_type=jnp.float32))
    o_ref[...] = jnp.where(h >= 0.0, h, 0.01 * h)


def _combine1(x, p, cnt, ws, wn):
    return pl.pallas_call(
        _combine_body,
        grid=(NBLK,),
        in_specs=[
            pl.BlockSpec((BLK, D), lambda i: (i, 0)),
            pl.BlockSpec((NC, BLK, D), lambda i: (0, i, 0)),
            pl.BlockSpec((NC, BLK, D), lambda i: (0, i, 0)),
            pl.BlockSpec((D, D), lambda i: (0, 0)),
            pl.BlockSpec((D, D), lambda i: (0, 0)),
        ],
        out_specs=pl.BlockSpec((BLK, D), lambda i: (i, 0)),
        out_shape=jax.ShapeDtypeStruct((NP, D), jnp.float32),
    )(x, p, cnt, ws, wn)


def _final_body(x_ref, p_ref, cnt_ref, ws_ref, wn_ref, batch_ref,
                w1_ref, b1_ref, w2_ref, b2_ref, o_ref, pacc, cacc):
    i = pl.program_id(0)
    ssum = p_ref[0] + p_ref[1]
    cntv = cnt_ref[0, :, 0:1] + cnt_ref[1, :, 0:1]
    agg = ssum / jnp.maximum(cntv, 1.0)
    h = (jnp.dot(x_ref[...], ws_ref[...], precision=_PREC,
                 preferred_element_type=jnp.float32)
         + jnp.dot(agg, wn_ref[...], precision=_PREC,
                   preferred_element_type=jnp.float32))
    h = jnp.where(h >= 0.0, h, 0.01 * h)
    ids = batch_ref[0]
    io = lax.broadcasted_iota(jnp.int32, (G, BLK), 0)
    oh = (io == ids).astype(jnp.float32)

    @pl.when(i == 0)
    def _():
        pacc[...] = jnp.zeros_like(pacc)
        cacc[...] = jnp.zeros_like(cacc)

    pacc[...] += jnp.dot(oh, h, precision=_PREC,
                         preferred_element_type=jnp.float32)
    cacc[...] += jnp.dot(oh, jnp.ones((BLK, D), jnp.float32), precision=_PREC,
                         preferred_element_type=jnp.float32)

    @pl.when(i == NBLK - 1)
    def _():
        pooled = pacc[...] / jnp.maximum(cacc[...], 1.0)
        t = jnp.dot(pooled, w1_ref[...], precision=_PREC,
                    preferred_element_type=jnp.float32) + b1_ref[...]
        t = jnp.maximum(t, 0.0)
        o_ref[...] = jnp.dot(t, w2_ref[...], precision=_PREC,
                             preferred_element_type=jnp.float32) + b2_ref[...]


def _final(x, p, cnt, ws, wn, batch3, w1, b1, w2, b2):
    return pl.pallas_call(
        _final_body,
        grid=(NBLK,),
        in_specs=[
            pl.BlockSpec((BLK, D), lambda i: (i, 0)),
            pl.BlockSpec((NC, BLK, D), lambda i: (0, i, 0)),
            pl.BlockSpec((NC, BLK, D), lambda i: (0, i, 0)),
            pl.BlockSpec((D, D), lambda i: (0, 0)),
            pl.BlockSpec((D, D), lambda i: (0, 0)),
            pl.BlockSpec((1, 1, BLK), lambda i: (i, 0, 0)),
            pl.BlockSpec((D, H), lambda i: (0, 0)),
            pl.BlockSpec((1, H), lambda i: (0, 0)),
            pl.BlockSpec((H, C), lambda i: (0, 0)),
            pl.BlockSpec((1, C), lambda i: (0, 0)),
        ],
        out_specs=pl.BlockSpec((G, C), lambda i: (0, 0)),
        out_shape=jax.ShapeDtypeStruct((G, C), jnp.float32),
        scratch_shapes=[
            pltpu.VMEM((G, D), jnp.float32),
            pltpu.VMEM((G, D), jnp.float32),
        ],
    )(x, p, cnt, ws, wn, batch3, w1, b1, w2, b2)


def kernel(x_lesion, edge_index, batch_lesion,
           W_self1, W_neigh1, W_self2, W_neigh2, W1, b1, W2, b2):
    x = jnp.concatenate(
        [x_lesion, jnp.zeros((NP - N, D), jnp.float32)], axis=0)
    src = edge_index[0].astype(jnp.int32).reshape(NC * NS, CHUNKS, K)
    dst = edge_index[1].astype(jnp.int32).reshape(NC * NS, CHUNKS, K)
    batch3 = jnp.concatenate(
        [batch_lesion.astype(jnp.int32),
         jnp.full((NP - N,), G, jnp.int32)]).reshape(NBLK, 1, BLK)
    zeros = jnp.zeros((NP, D), jnp.float32)
    ones = jnp.ones((K, D), jnp.float32)

    cnt = _make_sc_cnt()(dst, zeros, ones)
    p1 = _make_sc_agg()(x, src, dst, zeros)
    h2 = _combine1(x, p1, cnt, W_self1, W_neigh1)
    p2 = _make_sc_agg()(h2, src, dst, zeros)
    return _final(h2, p2, cnt, W_self2, W_neigh2, batch3,
                  W1, b1.reshape(1, H), W2, b2.reshape(1, C))


# speedup vs baseline: 7.6516x; 1.0034x over previous
import functools

import jax
import jax.numpy as jnp
from jax import lax
from jax.experimental import pallas as pl
from jax.experimental.pallas import tpu as pltpu
from jax.experimental.pallas import tpu_sc as plsc

N = 10000
NP = 10240
E = 320000
D = 128
G = 64
H = 64
C = 2

NC = 2
NS = 16
K = 125
TILE_EDGES = E // (NC * NS)
CHUNKS = TILE_EDGES // K
RPS = NP // NS
BLK = 1024
NBLK = NP // BLK

_PREC = lax.Precision.HIGHEST


def _sc_agg_body(x_hbm, src_hbm, dst_hbm, zeros_hbm,
                 p_ref, src_v, dst_v, rows2, gsem, acc):
    c = lax.axis_index("c")
    s = lax.axis_index("s")
    wid = c * NS + s
    r0 = s * RPS
    pltpu.sync_copy(zeros_hbm.at[pl.ds(r0, RPS)], acc.at[pl.ds(r0, RPS)])
    pltpu.sync_copy(src_hbm.at[wid], src_v)
    pltpu.sync_copy(dst_hbm.at[wid], dst_v)
    plsc.subcore_barrier()

    def step(j, carry):
        pltpu.async_copy(x_hbm.at[src_v.at[j]], rows2, gsem).wait()
        pltpu.sync_copy(rows2, acc.at[dst_v.at[j]], add=True)
        return carry

    lax.fori_loop(0, CHUNKS, step, 0)
    plsc.subcore_barrier()
    pltpu.sync_copy(acc.at[pl.ds(r0, RPS)], p_ref.at[c, pl.ds(r0, RPS)])


def _sc_aggcnt_body(x_hbm, src_hbm, dst_hbm, zeros_hbm, ones_hbm,
                    p_ref, cnt_ref, src_v, dst_v, rows2, gsem, acc):
    c = lax.axis_index("c")
    s = lax.axis_index("s")
    wid = c * NS + s
    r0 = s * RPS
    pltpu.sync_copy(zeros_hbm.at[pl.ds(r0, RPS)], acc.at[pl.ds(r0, RPS)])
    pltpu.sync_copy(src_hbm.at[wid], src_v)
    pltpu.sync_copy(dst_hbm.at[wid], dst_v)
    plsc.subcore_barrier()

    def phase_body(ph, carry):
        @pl.when(ph == 1)
        def _():
            plsc.subcore_barrier()
            pltpu.sync_copy(acc.at[pl.ds(r0, RPS)], p_ref.at[c, pl.ds(r0, RPS)])
            plsc.subcore_barrier()
            pltpu.sync_copy(zeros_hbm.at[pl.ds(r0, RPS)],
                            acc.at[pl.ds(r0, RPS)])
            pltpu.sync_copy(ones_hbm, rows2)
            plsc.subcore_barrier()

        def step(j, c2):
            @pl.when(ph == 0)
            def _():
                pltpu.async_copy(x_hbm.at[src_v.at[j]], rows2, gsem).wait()

            pltpu.sync_copy(rows2, acc.at[dst_v.at[j]], add=True)
            return c2

        lax.fori_loop(0, CHUNKS, step, 0)
        return carry

    lax.fori_loop(0, 2, phase_body, 0)
    plsc.subcore_barrier()
    pltpu.sync_copy(acc.at[pl.ds(r0, RPS)], cnt_ref.at[c, pl.ds(r0, RPS)])


def _make_sc_aggcnt():
    mesh = plsc.VectorSubcoreMesh(core_axis_name="c", subcore_axis_name="s")
    return pl.kernel(
        _sc_aggcnt_body,
        out_type=[jax.ShapeDtypeStruct((NC, NP, D), jnp.float32),
                  jax.ShapeDtypeStruct((NC, NP, D), jnp.float32)],
        mesh=mesh,
        scratch_types=[
            pltpu.VMEM((CHUNKS, K), jnp.int32),
            pltpu.VMEM((CHUNKS, K), jnp.int32),
            pltpu.VMEM((K, D), jnp.float32),
            pltpu.SemaphoreType.DMA,
            pltpu.VMEM_SHARED((NP, D), jnp.float32),
        ],
    )


def _sc_cnt_body(dst_hbm, zeros_hbm, ones_hbm,
                 cnt_ref, dst_v, ones_v, csem, cntacc):
    c = lax.axis_index("c")
    s = lax.axis_index("s")
    wid = c * NS + s
    r0 = s * RPS
    pltpu.sync_copy(zeros_hbm.at[pl.ds(r0, RPS)], cntacc.at[pl.ds(r0, RPS)])
    pltpu.sync_copy(ones_hbm, ones_v)
    pltpu.sync_copy(dst_hbm.at[wid], dst_v)
    plsc.subcore_barrier()

    def chunk(j, carry):
        @pl.when(j < CHUNKS)
        def _():
            pltpu.async_copy(ones_v, cntacc.at[dst_v.at[j]], csem, add=True)

        @pl.when(j >= 2)
        def _():
            pltpu.make_async_copy(ones_v, cntacc.at[dst_v.at[0]], csem).wait()

        return carry

    lax.fori_loop(0, CHUNKS + 2, chunk, 0)
    plsc.subcore_barrier()
    pltpu.sync_copy(cntacc.at[pl.ds(r0, RPS)], cnt_ref.at[c, pl.ds(r0, RPS)])


def _make_sc_agg():
    mesh = plsc.VectorSubcoreMesh(core_axis_name="c", subcore_axis_name="s")
    return pl.kernel(
        _sc_agg_body,
        out_type=jax.ShapeDtypeStruct((NC, NP, D), jnp.float32),
        mesh=mesh,
        scratch_types=[
            pltpu.VMEM((CHUNKS, K), jnp.int32),
            pltpu.VMEM((CHUNKS, K), jnp.int32),
            pltpu.VMEM((K, D), jnp.float32),
            pltpu.SemaphoreType.DMA,
            pltpu.VMEM_SHARED((NP, D), jnp.float32),
        ],
    )


def _make_sc_cnt():
    mesh = plsc.VectorSubcoreMesh(core_axis_name="c", subcore_axis_name="s")
    return pl.kernel(
        _sc_cnt_body,
        out_type=jax.ShapeDtypeStruct((NC, NP, D), jnp.float32),
        mesh=mesh,
        scratch_types=[
            pltpu.VMEM((CHUNKS, K), jnp.int32),
            pltpu.VMEM((K, D), jnp.float32),
            pltpu.SemaphoreType.DMA,
            pltpu.VMEM_SHARED((NP, D), jnp.float32),
        ],
    )


def _combine_body(x_ref, p_ref, cnt_ref, ws_ref, wn_ref, o_ref):
    ssum = p_ref[0] + p_ref[1]
    cntv = cnt_ref[0, :, 0:1] + cnt_ref[1, :, 0:1]
    agg = ssum / jnp.maximum(cntv, 1.0)
    h = (jnp.dot(x_ref[...], ws_ref[...], precision=_PREC,
                 preferred_element_type=jnp.float32)
         + jnp.dot(agg, wn_ref[...], precision=_PREC,
                   preferred_element_type=jnp.float32))
    o_ref[...] = jnp.where(h >= 0.0, h, 0.01 * h)


def _combine1(x, p, cnt, ws, wn):
    return pl.pallas_call(
        _combine_body,
        grid=(NBLK,),
        in_specs=[
            pl.BlockSpec((BLK, D), lambda i: (i, 0)),
            pl.BlockSpec((NC, BLK, D), lambda i: (0, i, 0)),
            pl.BlockSpec((NC, BLK, D), lambda i: (0, i, 0)),
            pl.BlockSpec((D, D), lambda i: (0, 0)),
            pl.BlockSpec((D, D), lambda i: (0, 0)),
        ],
        out_specs=pl.BlockSpec((BLK, D), lambda i: (i, 0)),
        out_shape=jax.ShapeDtypeStruct((NP, D), jnp.float32),
    )(x, p, cnt, ws, wn)


def _final_body(x_ref, p_ref, cnt_ref, ws_ref, wn_ref, batch_ref,
                w1_ref, b1_ref, w2_ref, b2_ref, o_ref, pacc, cacc):
    i = pl.program_id(0)
    ssum = p_ref[0] + p_ref[1]
    cntv = cnt_ref[0, :, 0:1] + cnt_ref[1, :, 0:1]
    agg = ssum / jnp.maximum(cntv, 1.0)
    h = (jnp.dot(x_ref[...], ws_ref[...], precision=_PREC,
                 preferred_element_type=jnp.float32)
         + jnp.dot(agg, wn_ref[...], precision=_PREC,
                   preferred_element_type=jnp.float32))
    h = jnp.where(h >= 0.0, h, 0.01 * h)
    ids = batch_ref[0]
    io = lax.broadcasted_iota(jnp.int32, (G, BLK), 0)
    oh = (io == ids).astype(jnp.float32)

    @pl.when(i == 0)
    def _():
        pacc[...] = jnp.zeros_like(pacc)
        cacc[...] = jnp.zeros_like(cacc)

    pacc[...] += jnp.dot(oh, h, precision=_PREC,
                         preferred_element_type=jnp.float32)
    cacc[...] += jnp.dot(oh, jnp.ones((BLK, D), jnp.float32), precision=_PREC,
                         preferred_element_type=jnp.float32)

    @pl.when(i == NBLK - 1)
    def _():
        pooled = pacc[...] / jnp.maximum(cacc[...], 1.0)
        t = jnp.dot(pooled, w1_ref[...], precision=_PREC,
                    preferred_element_type=jnp.float32) + b1_ref[...]
        t = jnp.maximum(t, 0.0)
        o_ref[...] = jnp.dot(t, w2_ref[...], precision=_PREC,
                             preferred_element_type=jnp.float32) + b2_ref[...]


def _final(x, p, cnt, ws, wn, batch3, w1, b1, w2, b2):
    return pl.pallas_call(
        _final_body,
        grid=(NBLK,),
        in_specs=[
            pl.BlockSpec((BLK, D), lambda i: (i, 0)),
            pl.BlockSpec((NC, BLK, D), lambda i: (0, i, 0)),
            pl.BlockSpec((NC, BLK, D), lambda i: (0, i, 0)),
            pl.BlockSpec((D, D), lambda i: (0, 0)),
            pl.BlockSpec((D, D), lambda i: (0, 0)),
            pl.BlockSpec((1, 1, BLK), lambda i: (i, 0, 0)),
            pl.BlockSpec((D, H), lambda i: (0, 0)),
            pl.BlockSpec((1, H), lambda i: (0, 0)),
            pl.BlockSpec((H, C), lambda i: (0, 0)),
            pl.BlockSpec((1, C), lambda i: (0, 0)),
        ],
        out_specs=pl.BlockSpec((G, C), lambda i: (0, 0)),
        out_shape=jax.ShapeDtypeStruct((G, C), jnp.float32),
        scratch_shapes=[
            pltpu.VMEM((G, D), jnp.float32),
            pltpu.VMEM((G, D), jnp.float32),
        ],
    )(x, p, cnt, ws, wn, batch3, w1, b1, w2, b2)


def kernel(x_lesion, edge_index, batch_lesion,
           W_self1, W_neigh1, W_self2, W_neigh2, W1, b1, W2, b2):
    x = jnp.concatenate(
        [x_lesion, jnp.zeros((NP - N, D), jnp.float32)], axis=0)
    src = edge_index[0].astype(jnp.int32).reshape(NC * NS, CHUNKS, K)
    dst = edge_index[1].astype(jnp.int32).reshape(NC * NS, CHUNKS, K)
    batch3 = jnp.concatenate(
        [batch_lesion.astype(jnp.int32),
         jnp.full((NP - N,), G, jnp.int32)]).reshape(NBLK, 1, BLK)
    zeros = jnp.zeros((NP, D), jnp.float32)
    ones = jnp.ones((K, D), jnp.float32)

    p1, cnt = _make_sc_aggcnt()(x, src, dst, zeros, ones)
    h2 = _combine1(x, p1, cnt, W_self1, W_neigh1)
    p2 = _make_sc_agg()(h2, src, dst, zeros)
    return _final(h2, p2, cnt, W_self2, W_neigh2, batch3,
                  W1, b1.reshape(1, H), W2, b2.reshape(1, C))
